# Initial kernel scaffold; baseline (speedup 1.0000x reference)
import jax
import jax.numpy as jnp
from jax.experimental import pallas as pl


def kernel(node_emb, cx, cat_table, W_ih, W_hh, b_ih, b_hh, edge_src, edge_dst, edge_cat, seed_idx, ns_idx):
    raise NotImplementedError("write your pallas kernel here")


# speedup vs baseline: 1.0047x; 1.0047x over previous
import functools

import jax
import jax.numpy as jnp
from jax import lax
from jax.experimental import pallas as pl
from jax.experimental.pallas import tpu as pltpu

N_NODES = 50000
USER_NUM = 40000
CAT_NUM = 18
EMB = 100
EMBP = 128
B_SEED = 4096
N_CAND = 9
N_CANDP = 16
BLK = 512


def _lstm_loss_kernel(x_ref, h_ref, c_ref, wih_ref, whh_ref, b_ref, cand_ref,
                      out_ref):
    i = pl.program_id(0)
    x = x_ref[...]
    h = h_ref[...]
    gates = (lax.dot_general(x, wih_ref[...], (((1,), (1,)), ((), ())),
                             preferred_element_type=jnp.float32)
             + lax.dot_general(h, whh_ref[...], (((1,), (1,)), ((), ())),
                               preferred_element_type=jnp.float32)
             + b_ref[...])
    i_g = jax.nn.sigmoid(gates[:, 0:EMBP])
    f_g = jax.nn.sigmoid(gates[:, EMBP:2 * EMBP])
    g_g = jnp.tanh(gates[:, 2 * EMBP:3 * EMBP])
    o_g = jax.nn.sigmoid(gates[:, 3 * EMBP:4 * EMBP])
    c_new = f_g * c_ref[...] + i_g * g_g
    h_new = o_g * jnp.tanh(c_new)
    score = jnp.sum(h_new[:, None, :] * cand_ref[...], axis=-1)
    col = lax.broadcasted_iota(jnp.int32, score.shape, 1)
    score = jnp.where(col < N_CAND, score, -1e30)
    m = jnp.max(score, axis=-1, keepdims=True)
    lse = m[:, 0] + jnp.log(jnp.sum(jnp.exp(score - m), axis=-1))
    part = jnp.sum(score[:, 0] - lse).reshape(1, 1)

    @pl.when(i == 0)
    def _():
        out_ref[...] = jnp.zeros((1, 1), jnp.float32)

    out_ref[...] += part


def _lstm_loss(x, h, c, wih, whh, b, cand):
    nblk = B_SEED // BLK
    out = pl.pallas_call(
        _lstm_loss_kernel,
        grid=(nblk,),
        in_specs=[
            pl.BlockSpec((BLK, EMBP), lambda i: (i, 0)),
            pl.BlockSpec((BLK, EMBP), lambda i: (i, 0)),
            pl.BlockSpec((BLK, EMBP), lambda i: (i, 0)),
            pl.BlockSpec((4 * EMBP, EMBP), lambda i: (0, 0)),
            pl.BlockSpec((4 * EMBP, EMBP), lambda i: (0, 0)),
            pl.BlockSpec((1, 4 * EMBP), lambda i: (0, 0)),
            pl.BlockSpec((BLK, N_CANDP, EMBP), lambda i: (i, 0, 0)),
        ],
        out_specs=pl.BlockSpec((1, 1), lambda i: (0, 0)),
        out_shape=jax.ShapeDtypeStruct((1, 1), jnp.float32),
    )(x, h, c, wih, whh, b, cand)
    return out[0, 0] * (-1.0 / B_SEED)


def _pad_gate_w(w):
    w = w.reshape(4, EMB, EMB)
    w = jnp.pad(w, ((0, 0), (0, EMBP - EMB), (0, EMBP - EMB)))
    return w.reshape(4 * EMBP, EMBP)


def kernel(node_emb, cx, cat_table, W_ih, W_hh, b_ih, b_hh,
           edge_src, edge_dst, edge_cat, seed_idx, ns_idx):
    msg = node_emb[edge_src] * cat_table[edge_cat]
    agg = jax.ops.segment_sum(msg, edge_dst, num_segments=N_NODES)
    deg = jax.ops.segment_sum(jnp.ones_like(edge_dst, jnp.float32), edge_dst,
                              num_segments=N_NODES)
    node_emb2 = node_emb + agg / jnp.clip(deg, 1.0)[:, None]

    x = node_emb2[seed_idx]
    h = node_emb[seed_idx]
    c = cx[seed_idx]
    cand = node_emb2[ns_idx + USER_NUM]

    pad = ((0, 0), (0, EMBP - EMB))
    x = jnp.pad(x, pad)
    h = jnp.pad(h, pad)
    c = jnp.pad(c, pad)
    cand = jnp.pad(cand, ((0, 0), (0, N_CANDP - N_CAND), (0, EMBP - EMB)))
    wih = _pad_gate_w(W_ih)
    whh = _pad_gate_w(W_hh)
    b = (b_ih + b_hh).reshape(4, EMB)
    b = jnp.pad(b, ((0, 0), (0, EMBP - EMB))).reshape(1, 4 * EMBP)
    return _lstm_loss(x, h, c, wih, whh, b, cand)


# speedup vs baseline: 1.2111x; 1.2054x over previous
import functools

import jax
import jax.numpy as jnp
from jax import lax
from jax.experimental import pallas as pl
from jax.experimental.pallas import tpu as pltpu
from jax.experimental.pallas import tpu_sc as plsc

N_NODES = 50000
USER_NUM = 40000
CAT_NUM = 18
EMB = 100
EMBP = 128
B_SEED = 4096
N_CAND = 9
N_CANDP = 16
BLK = 512

NQ = 8
QB = (0, 6256, 12512, 18768, 25024, 31280, 37536, 43792, 50000)
QPAD = 6400
DUMP = 6300
CW = 128
DEGC = 100
EC = 128
NP = 50176
N_OUT_CHUNKS = 49

_i32 = jnp.int32
_f32 = jnp.float32


def _mp_body(emb_hbm, src_hbm, dst_hbm, ecat_hbm, ctab_hbm, offs_hbm, out_hbm,
             agg, offs_v, zero_b, rows_b, emb_b, srci_b, dst_b,
             cat_b, dstloc_b, ctab_v, gsem):
    c = lax.axis_index("c")
    s = lax.axis_index("s")
    iota = lax.iota(_i32, 16)

    pltpu.sync_copy(ctab_hbm, ctab_v)
    pltpu.sync_copy(offs_hbm, offs_v)

    def ext(j):
        v = offs_v[pl.ds((j // 16) * 16, 16)]
        return jnp.max(jnp.where(iota == (j % 16), v, 0))

    zvec = jnp.zeros((16,), _f32)

    def zrow(i, carry):
        for cc in range(CW // 16):
            zero_b[i, pl.ds(cc * 16, 16)] = zvec
        return carry

    lax.fori_loop(0, 16, zrow, 0)

    for r in range(NQ // 2):
        q = (NQ // 2) * c + r
        qbase = pl.multiple_of(ext(16 + q), 8)

        zb = s * (QPAD // 16)
        for j in range(QPAD // 16 // 16):
            pltpu.sync_copy(zero_b, agg.at[pl.ds(zb + j * 16, 16)])
        plsc.subcore_barrier()

        lo = ext(q)
        hi = ext(q + 1)
        cnt = hi - lo
        my_lo = lo + (cnt * s) // 16
        my_hi = lo + (cnt * (s + 1)) // 16
        my_lo_al = pl.multiple_of((my_lo // 8) * 8, 8)
        nch = (my_hi - my_lo_al + EC - 1) // EC

        def echunk(j, carry):
            base = my_lo_al + j * EC
            pltpu.sync_copy(src_hbm.at[pl.ds(base, EC)], srci_b)
            pltpu.sync_copy(dst_hbm.at[pl.ds(base, EC)], dst_b)
            pltpu.sync_copy(ecat_hbm.at[pl.ds(base, EC)], cat_b)
            pltpu.async_copy(emb_hbm.at[srci_b], rows_b, gsem).wait()
            for g in range(EC // 16):
                e_v = g * 16 + iota
                pos = base + e_v
                dstv = dst_b[pl.ds(g * 16, 16)]
                catv = cat_b[pl.ds(g * 16, 16)]
                valid = (pos >= my_lo) & (pos < my_hi)
                dstloc_b[0, pl.ds(g * 16, 16)] = jnp.where(
                    valid, dstv - qbase, DUMP)

                def colbody(k, carry2):
                    fk = jnp.full((16,), k, _i32)
                    rv = plsc.load_gather(rows_b, [e_v, fk])
                    cv = plsc.load_gather(ctab_v, [catv, fk])
                    plsc.store_scatter(rows_b, [e_v, fk], rv * cv)
                    return carry2

                lax.fori_loop(0, EMB, colbody, 0)
            pltpu.sync_copy(rows_b, agg.at[dstloc_b.at[0]], add=True)
            return carry

        lax.fori_loop(0, nch, echunk, 0)
        plsc.subcore_barrier()

        for j in range(4):
            m = s + 16 * j

            @pl.when(m < N_OUT_CHUNKS)
            def _chunk(m=m):
                rb = m * 128
                pltpu.sync_copy(agg.at[pl.ds(rb, 128)], rows_b)
                pltpu.sync_copy(emb_hbm.at[pl.ds(qbase + rb, 128)], emb_b)
                degcol = jnp.full((16,), DEGC, _i32)
                for g in range(8):
                    e_v = g * 16 + iota
                    degv = plsc.load_gather(rows_b, [e_v, degcol])
                    rinv = 1.0 / jnp.maximum(degv, 1.0)

                    def ocol(k, carry2):
                        fk = jnp.full((16,), k, _i32)
                        av = plsc.load_gather(rows_b, [e_v, fk])
                        bv = plsc.load_gather(emb_b, [e_v, fk])
                        plsc.store_scatter(rows_b, [e_v, fk], bv + av * rinv)
                        return carry2

                    lax.fori_loop(0, EMB, ocol, 0)

                pltpu.sync_copy(rows_b, out_hbm.at[q, pl.ds(rb, 128)])

        plsc.subcore_barrier()


_mp_kernel = pl.kernel(
    _mp_body,
    out_type=jax.ShapeDtypeStruct((NQ, QPAD, CW), _f32),
    mesh=plsc.VectorSubcoreMesh(core_axis_name="c", subcore_axis_name="s"),
    compiler_params=pltpu.CompilerParams(needs_layout_passes=False),
    scratch_types=[
        pltpu.VMEM_SHARED((QPAD, CW), _f32),
        pltpu.VMEM((32,), _i32),
        pltpu.VMEM((16, CW), _f32),
        pltpu.VMEM((EC, CW), _f32),
        pltpu.VMEM((128, CW), _f32),
        pltpu.VMEM((EC,), _i32),
        pltpu.VMEM((EC,), _i32),
        pltpu.VMEM((EC,), _i32),
        pltpu.VMEM((1, EC), _i32),
        pltpu.VMEM((CAT_NUM, EMB), _f32),
        pltpu.SemaphoreType.DMA,
    ],
)


def _lstm_loss_kernel(x_ref, h_ref, c_ref, wih_ref, whh_ref, b_ref, cand_ref,
                      out_ref):
    i = pl.program_id(0)
    x = x_ref[...]
    h = h_ref[...]
    gates = (lax.dot_general(x, wih_ref[...], (((1,), (1,)), ((), ())),
                             preferred_element_type=jnp.float32)
             + lax.dot_general(h, whh_ref[...], (((1,), (1,)), ((), ())),
                               preferred_element_type=jnp.float32)
             + b_ref[...])
    i_g = jax.nn.sigmoid(gates[:, 0:EMBP])
    f_g = jax.nn.sigmoid(gates[:, EMBP:2 * EMBP])
    g_g = jnp.tanh(gates[:, 2 * EMBP:3 * EMBP])
    o_g = jax.nn.sigmoid(gates[:, 3 * EMBP:4 * EMBP])
    c_new = f_g * c_ref[...] + i_g * g_g
    h_new = o_g * jnp.tanh(c_new)
    score = jnp.sum(h_new[:, None, :] * cand_ref[...], axis=-1)
    col = lax.broadcasted_iota(jnp.int32, score.shape, 1)
    score = jnp.where(col < N_CAND, score, -1e30)
    m = jnp.max(score, axis=-1, keepdims=True)
    lse = m[:, 0] + jnp.log(jnp.sum(jnp.exp(score - m), axis=-1))
    part = jnp.sum(score[:, 0] - lse).reshape(1, 1)

    @pl.when(i == 0)
    def _():
        out_ref[...] = jnp.zeros((1, 1), jnp.float32)

    out_ref[...] += part


def _lstm_loss(x, h, c, wih, whh, b, cand):
    nblk = B_SEED // BLK
    out = pl.pallas_call(
        _lstm_loss_kernel,
        grid=(nblk,),
        in_specs=[
            pl.BlockSpec((BLK, EMBP), lambda i: (i, 0)),
            pl.BlockSpec((BLK, EMBP), lambda i: (i, 0)),
            pl.BlockSpec((BLK, EMBP), lambda i: (i, 0)),
            pl.BlockSpec((4 * EMBP, EMBP), lambda i: (0, 0)),
            pl.BlockSpec((4 * EMBP, EMBP), lambda i: (0, 0)),
            pl.BlockSpec((1, 4 * EMBP), lambda i: (0, 0)),
            pl.BlockSpec((BLK, N_CANDP, EMBP), lambda i: (i, 0, 0)),
        ],
        out_specs=pl.BlockSpec((1, 1), lambda i: (0, 0)),
        out_shape=jax.ShapeDtypeStruct((1, 1), jnp.float32),
    )(x, h, c, wih, whh, b, cand)
    return out[0, 0] * (-1.0 / B_SEED)


def _pad_gate_w(w):
    w = w.reshape(4, EMB, EMB)
    w = jnp.pad(w, ((0, 0), (0, EMBP - EMB), (0, EMBP - EMB)))
    return w.reshape(4 * EMBP, EMBP)


def kernel(node_emb, cx, cat_table, W_ih, W_hh, b_ih, b_hh,
           edge_src, edge_dst, edge_cat, seed_idx, ns_idx):
    emb_pad = jnp.zeros((NP, CW), _f32)
    emb_pad = emb_pad.at[:N_NODES, :EMB].set(node_emb)
    emb_pad = emb_pad.at[:N_NODES, DEGC].set(1.0)
    src_p = jnp.pad(edge_src.astype(_i32), (0, 256))
    dst_p = jnp.pad(edge_dst.astype(_i32), (0, 256))
    cat_p = jnp.pad(edge_cat.astype(_i32), (0, 256))
    qb = jnp.array(QB, _i32)
    eoffs = jnp.searchsorted(edge_dst, qb).astype(_i32)
    offs = jnp.zeros((32,), _i32).at[0:NQ + 1].set(eoffs).at[16:16 + NQ + 1].set(qb)

    node2 = _mp_kernel(emb_pad, src_p, dst_p, cat_p,
                       cat_table.astype(_f32), offs)
    node_emb2 = jnp.concatenate(
        [node2[i, :QB[i + 1] - QB[i], :EMB] for i in range(NQ)], axis=0)

    x = node_emb2[seed_idx]
    h = node_emb[seed_idx]
    c = cx[seed_idx]
    cand = node_emb2[ns_idx + USER_NUM]

    pad = ((0, 0), (0, EMBP - EMB))
    x = jnp.pad(x, pad)
    h = jnp.pad(h, pad)
    c = jnp.pad(c, pad)
    cand = jnp.pad(cand, ((0, 0), (0, N_CANDP - N_CAND), (0, EMBP - EMB)))
    wih = _pad_gate_w(W_ih)
    whh = _pad_gate_w(W_hh)
    b = (b_ih + b_hh).reshape(4, EMB)
    b = jnp.pad(b, ((0, 0), (0, EMBP - EMB))).reshape(1, 4 * EMBP)
    return _lstm_loss(x, h, c, wih, whh, b, cand)


# speedup vs baseline: 2.5723x; 2.1240x over previous
import functools

import jax
import jax.numpy as jnp
from jax import lax
from jax.experimental import pallas as pl
from jax.experimental.pallas import tpu as pltpu
from jax.experimental.pallas import tpu_sc as plsc

N_NODES = 50000
USER_NUM = 40000
CAT_NUM = 18
EMB = 100
EMBP = 128
B_SEED = 4096
N_CAND = 9
N_CANDP = 16
BLK = 512

NQ = 8
QB = (0, 6256, 12512, 18768, 25024, 31280, 37536, 43792, 50000)
QPAD = 6400
DUMP = 6300
CW = 128
DEGC = 100
EC = 128
NP = 50176
N_OUT_CHUNKS = 49

_i32 = jnp.int32
_f32 = jnp.float32


def _mp_body(emb_hbm, src_hbm, dst_hbm, ecat_hbm, ctab_hbm, offs_hbm, out_hbm,
             agg, offs_v, zero_b, rows_b, crows_b, emb_b, srci_b, dst_b,
             cat_b, dstloc_b, gsem):
    c = lax.axis_index("c")
    s = lax.axis_index("s")
    iota = lax.iota(_i32, 16)

    pltpu.sync_copy(offs_hbm, offs_v)

    def ext(j):
        v = offs_v[pl.ds((j // 16) * 16, 16)]
        return jnp.max(jnp.where(iota == (j % 16), v, 0))

    zvec = jnp.zeros((16,), _f32)

    def zrow(i, carry):
        for cc in range(CW // 16):
            zero_b[i, pl.ds(cc * 16, 16)] = zvec
        return carry

    lax.fori_loop(0, 16, zrow, 0)

    for r in range(NQ // 2):
        q = (NQ // 2) * c + r
        qbase = pl.multiple_of(ext(16 + q), 8)

        zb = s * (QPAD // 16)
        for j in range(QPAD // 16 // 16):
            pltpu.sync_copy(zero_b, agg.at[pl.ds(zb + j * 16, 16)])
        plsc.subcore_barrier()

        lo = ext(q)
        hi = ext(q + 1)
        cnt = hi - lo
        my_lo = lo + (cnt * s) // 16
        my_hi = lo + (cnt * (s + 1)) // 16
        my_lo_al = pl.multiple_of((my_lo // 8) * 8, 8)
        nch = (my_hi - my_lo_al + EC - 1) // EC

        def echunk(j, carry):
            base = my_lo_al + j * EC
            pltpu.sync_copy(src_hbm.at[pl.ds(base, EC)], srci_b)
            pltpu.sync_copy(dst_hbm.at[pl.ds(base, EC)], dst_b)
            pltpu.sync_copy(ecat_hbm.at[pl.ds(base, EC)], cat_b)
            cp1 = pltpu.async_copy(emb_hbm.at[srci_b], rows_b, gsem)
            cp2 = pltpu.async_copy(ctab_hbm.at[cat_b], crows_b, gsem)
            for g in range(EC // 16):
                e_v = g * 16 + iota
                pos = base + e_v
                dstv = dst_b[pl.ds(g * 16, 16)]
                valid = (pos >= my_lo) & (pos < my_hi)
                dstloc_b[0, pl.ds(g * 16, 16)] = jnp.where(
                    valid, dstv - qbase, DUMP)
            cp1.wait()
            cp2.wait()

            def mrow(i, carry2):
                for cc in range(CW // 16):
                    sl = pl.ds(cc * 16, 16)
                    rows_b[i, sl] = rows_b[i, sl] * crows_b[i, sl]
                return carry2

            lax.fori_loop(0, EC, mrow, 0)
            pltpu.sync_copy(rows_b, agg.at[dstloc_b.at[0]], add=True)
            return carry

        lax.fori_loop(0, nch, echunk, 0)
        plsc.subcore_barrier()

        for j in range(4):
            m = s + 16 * j

            @pl.when(m < N_OUT_CHUNKS)
            def _chunk(m=m):
                rb = m * 128
                pltpu.sync_copy(agg.at[pl.ds(rb, 128)], rows_b)
                pltpu.sync_copy(emb_hbm.at[pl.ds(qbase + rb, 128)], emb_b)
                degcol = jnp.full((16,), DEGC, _i32)
                for g in range(8):
                    e_v = g * 16 + iota
                    degv = plsc.load_gather(rows_b, [e_v, degcol])
                    rinv = 1.0 / jnp.maximum(degv, 1.0)

                    def ocol(k, carry2):
                        fk = jnp.full((16,), k, _i32)
                        av = plsc.load_gather(rows_b, [e_v, fk])
                        bv = plsc.load_gather(emb_b, [e_v, fk])
                        plsc.store_scatter(rows_b, [e_v, fk], bv + av * rinv)
                        return carry2

                    lax.fori_loop(0, EMB, ocol, 0)

                pltpu.sync_copy(rows_b, out_hbm.at[q, pl.ds(rb, 128)])

        plsc.subcore_barrier()


_mp_kernel = pl.kernel(
    _mp_body,
    out_type=jax.ShapeDtypeStruct((NQ, QPAD, CW), _f32),
    mesh=plsc.VectorSubcoreMesh(core_axis_name="c", subcore_axis_name="s"),
    compiler_params=pltpu.CompilerParams(needs_layout_passes=False),
    scratch_types=[
        pltpu.VMEM_SHARED((QPAD, CW), _f32),
        pltpu.VMEM((32,), _i32),
        pltpu.VMEM((16, CW), _f32),
        pltpu.VMEM((EC, CW), _f32),
        pltpu.VMEM((EC, CW), _f32),
        pltpu.VMEM((128, CW), _f32),
        pltpu.VMEM((EC,), _i32),
        pltpu.VMEM((EC,), _i32),
        pltpu.VMEM((EC,), _i32),
        pltpu.VMEM((1, EC), _i32),
        pltpu.SemaphoreType.DMA,
    ],
)


def _lstm_loss_kernel(x_ref, h_ref, c_ref, wih_ref, whh_ref, b_ref, cand_ref,
                      out_ref):
    i = pl.program_id(0)
    x = x_ref[...]
    h = h_ref[...]
    gates = (lax.dot_general(x, wih_ref[...], (((1,), (1,)), ((), ())),
                             preferred_element_type=jnp.float32)
             + lax.dot_general(h, whh_ref[...], (((1,), (1,)), ((), ())),
                               preferred_element_type=jnp.float32)
             + b_ref[...])
    i_g = jax.nn.sigmoid(gates[:, 0:EMBP])
    f_g = jax.nn.sigmoid(gates[:, EMBP:2 * EMBP])
    g_g = jnp.tanh(gates[:, 2 * EMBP:3 * EMBP])
    o_g = jax.nn.sigmoid(gates[:, 3 * EMBP:4 * EMBP])
    c_new = f_g * c_ref[...] + i_g * g_g
    h_new = o_g * jnp.tanh(c_new)
    score = jnp.sum(h_new[:, None, :] * cand_ref[...], axis=-1)
    col = lax.broadcasted_iota(jnp.int32, score.shape, 1)
    score = jnp.where(col < N_CAND, score, -1e30)
    m = jnp.max(score, axis=-1, keepdims=True)
    lse = m[:, 0] + jnp.log(jnp.sum(jnp.exp(score - m), axis=-1))
    part = jnp.sum(score[:, 0] - lse).reshape(1, 1)

    @pl.when(i == 0)
    def _():
        out_ref[...] = jnp.zeros((1, 1), jnp.float32)

    out_ref[...] += part


def _lstm_loss(x, h, c, wih, whh, b, cand):
    nblk = B_SEED // BLK
    out = pl.pallas_call(
        _lstm_loss_kernel,
        grid=(nblk,),
        in_specs=[
            pl.BlockSpec((BLK, EMBP), lambda i: (i, 0)),
            pl.BlockSpec((BLK, EMBP), lambda i: (i, 0)),
            pl.BlockSpec((BLK, EMBP), lambda i: (i, 0)),
            pl.BlockSpec((4 * EMBP, EMBP), lambda i: (0, 0)),
            pl.BlockSpec((4 * EMBP, EMBP), lambda i: (0, 0)),
            pl.BlockSpec((1, 4 * EMBP), lambda i: (0, 0)),
            pl.BlockSpec((BLK, N_CANDP, EMBP), lambda i: (i, 0, 0)),
        ],
        out_specs=pl.BlockSpec((1, 1), lambda i: (0, 0)),
        out_shape=jax.ShapeDtypeStruct((1, 1), jnp.float32),
    )(x, h, c, wih, whh, b, cand)
    return out[0, 0] * (-1.0 / B_SEED)


def _pad_gate_w(w):
    w = w.reshape(4, EMB, EMB)
    w = jnp.pad(w, ((0, 0), (0, EMBP - EMB), (0, EMBP - EMB)))
    return w.reshape(4 * EMBP, EMBP)


def kernel(node_emb, cx, cat_table, W_ih, W_hh, b_ih, b_hh,
           edge_src, edge_dst, edge_cat, seed_idx, ns_idx):
    emb_pad = jnp.zeros((NP, CW), _f32)
    emb_pad = emb_pad.at[:N_NODES, :EMB].set(node_emb)
    emb_pad = emb_pad.at[:N_NODES, DEGC].set(1.0)
    src_p = jnp.pad(edge_src.astype(_i32), (0, 256))
    dst_p = jnp.pad(edge_dst.astype(_i32), (0, 256))
    cat_p = jnp.pad(edge_cat.astype(_i32), (0, 256))
    qb = jnp.array(QB, _i32)
    eoffs = jnp.searchsorted(edge_dst, qb).astype(_i32)
    offs = jnp.zeros((32,), _i32).at[0:NQ + 1].set(eoffs).at[16:16 + NQ + 1].set(qb)

    ctab_pad = jnp.zeros((32, CW), _f32)
    ctab_pad = ctab_pad.at[:CAT_NUM, :EMB].set(cat_table.astype(_f32))
    ctab_pad = ctab_pad.at[:CAT_NUM, DEGC].set(1.0)

    node2 = _mp_kernel(emb_pad, src_p, dst_p, cat_p, ctab_pad, offs)
    node_emb2 = jnp.concatenate(
        [node2[i, :QB[i + 1] - QB[i], :EMB] for i in range(NQ)], axis=0)

    x = node_emb2[seed_idx]
    h = node_emb[seed_idx]
    c = cx[seed_idx]
    cand = node_emb2[ns_idx + USER_NUM]

    pad = ((0, 0), (0, EMBP - EMB))
    x = jnp.pad(x, pad)
    h = jnp.pad(h, pad)
    c = jnp.pad(c, pad)
    cand = jnp.pad(cand, ((0, 0), (0, N_CANDP - N_CAND), (0, EMBP - EMB)))
    wih = _pad_gate_w(W_ih)
    whh = _pad_gate_w(W_hh)
    b = (b_ih + b_hh).reshape(4, EMB)
    b = jnp.pad(b, ((0, 0), (0, EMBP - EMB))).reshape(1, 4 * EMBP)
    return _lstm_loss(x, h, c, wih, whh, b, cand)


# speedup vs baseline: 2.5862x; 1.0054x over previous
import functools

import jax
import jax.numpy as jnp
from jax import lax
from jax.experimental import pallas as pl
from jax.experimental.pallas import tpu as pltpu
from jax.experimental.pallas import tpu_sc as plsc

N_NODES = 50000
USER_NUM = 40000
CAT_NUM = 18
EMB = 100
EMBP = 128
B_SEED = 4096
N_CAND = 9
N_CANDP = 16
BLK = 512

NQ = 10
QB = (0, 5008, 10016, 15024, 20032, 25040, 30048, 35056, 40064, 45072, 50000)
QPAD = 5120
DUMP = 5050
CW = 128
DEGC = 100
EC = 128
NP = 50176
N_OUT_CHUNKS = 40

_i32 = jnp.int32
_f32 = jnp.float32


def _mp_body(emb_hbm, src_hbm, dst_hbm, ecat_hbm, ctab_hbm, offs_hbm, out_hbm,
             agg, offs_v, zero_b, rows_b0, rows_b1, crows_b0, crows_b1, emb_b,
             srci_b0, srci_b1, dst_b0, dst_b1, cat_b0, cat_b1, dstloc_b,
             isem0, isem1, gsem0, gsem1):
    rows_b = (rows_b0, rows_b1)
    crows_b = (crows_b0, crows_b1)
    srci_b = (srci_b0, srci_b1)
    dst_b = (dst_b0, dst_b1)
    cat_b = (cat_b0, cat_b1)
    isem = (isem0, isem1)
    gsem = (gsem0, gsem1)
    c = lax.axis_index("c")
    s = lax.axis_index("s")
    iota = lax.iota(_i32, 16)

    pltpu.sync_copy(offs_hbm, offs_v)

    def ext(j):
        v = offs_v[pl.ds((j // 16) * 16, 16)]
        return jnp.max(jnp.where(iota == (j % 16), v, 0))

    zvec = jnp.zeros((16,), _f32)

    def zrow(i, carry):
        for cc in range(CW // 16):
            zero_b[i, pl.ds(cc * 16, 16)] = zvec
        return carry

    lax.fori_loop(0, 16, zrow, 0)

    for r in range(NQ // 2):
        q = (NQ // 2) * c + r
        qbase = pl.multiple_of(ext(16 + q), 8)

        zb = s * (QPAD // 16)
        for j in range(QPAD // 16 // 16):
            pltpu.sync_copy(zero_b, agg.at[pl.ds(zb + j * 16, 16)])
        plsc.subcore_barrier()

        lo = ext(q)
        hi = ext(q + 1)
        cnt = hi - lo
        my_lo = lo + (cnt * s) // 16
        my_hi = lo + (cnt * (s + 1)) // 16
        my_lo_al = pl.multiple_of((my_lo // 8) * 8, 8)
        nch = (my_hi - my_lo_al + EC - 1) // EC
        ncl = jnp.maximum(nch, 1)

        def cbase(j):
            return my_lo_al + jnp.minimum(j, ncl - 1) * EC

        def issue_idx(j, b):
            base = pl.multiple_of(cbase(j), 8)
            pltpu.async_copy(src_hbm.at[pl.ds(base, EC)], srci_b[b], isem[b])
            pltpu.async_copy(dst_hbm.at[pl.ds(base, EC)], dst_b[b], isem[b])
            pltpu.async_copy(ecat_hbm.at[pl.ds(base, EC)], cat_b[b], isem[b])

        def wait_idx(b):
            pltpu.make_async_copy(src_hbm.at[pl.ds(0, EC)], srci_b[b],
                                  isem[b]).wait()
            pltpu.make_async_copy(dst_hbm.at[pl.ds(0, EC)], dst_b[b],
                                  isem[b]).wait()
            pltpu.make_async_copy(ecat_hbm.at[pl.ds(0, EC)], cat_b[b],
                                  isem[b]).wait()

        def issue_gather(b):
            pltpu.async_copy(emb_hbm.at[srci_b[b]], rows_b[b], gsem[b])
            pltpu.async_copy(ctab_hbm.at[cat_b[b]], crows_b[b], gsem[b])

        def wait_gather(b):
            pltpu.make_async_copy(emb_hbm.at[srci_b[b]], rows_b[b],
                                  gsem[b]).wait()
            pltpu.make_async_copy(ctab_hbm.at[cat_b[b]], crows_b[b],
                                  gsem[b]).wait()

        def calc_dstloc(j, b):
            base = my_lo_al + j * EC
            for g in range(EC // 16):
                e_v = g * 16 + iota
                pos = base + e_v
                dstv = dst_b[b][pl.ds(g * 16, 16)]
                valid = (pos >= my_lo) & (pos < my_hi)
                dstloc_b[0, pl.ds(g * 16, 16)] = jnp.where(
                    valid, dstv - qbase, DUMP)

        def mul_scatter(b):
            def mrow(i, carry2):
                for cc in range(CW // 16):
                    sl = pl.ds(cc * 16, 16)
                    rows_b[b][i, sl] = rows_b[b][i, sl] * crows_b[b][i, sl]
                return carry2

            lax.fori_loop(0, EC, mrow, 0)
            pltpu.sync_copy(rows_b[b], agg.at[dstloc_b.at[0]], add=True)

        issue_idx(0, 0)
        issue_idx(1, 1)
        wait_idx(0)
        issue_gather(0)

        def pipebody(g, carry):
            j0 = 2 * g
            j1 = j0 + 1
            wait_idx(1)
            issue_gather(1)
            wait_gather(0)
            calc_dstloc(j0, 0)
            issue_idx(j0 + 2, 0)
            mul_scatter(0)
            wait_idx(0)
            wait_gather(1)
            issue_gather(0)
            calc_dstloc(j1, 1)
            issue_idx(j1 + 2, 1)
            mul_scatter(1)
            return carry

        npair = (ncl + 1) // 2
        lax.fori_loop(0, npair - 1, pipebody, 0)

        t2 = 2 * npair - 2
        wait_idx(1)
        issue_gather(1)
        wait_gather(0)
        calc_dstloc(t2, 0)
        mul_scatter(0)
        wait_gather(1)
        calc_dstloc(t2 + 1, 1)
        mul_scatter(1)
        plsc.subcore_barrier()

        for j in range(3):
            m = s + 16 * j

            @pl.when(m < N_OUT_CHUNKS)
            def _chunk(m=m):
                rb = m * 128
                pltpu.sync_copy(agg.at[pl.ds(rb, 128)], rows_b0)
                pltpu.sync_copy(emb_hbm.at[pl.ds(qbase + rb, 128)], emb_b)
                degcol = jnp.full((16,), DEGC, _i32)
                for g in range(8):
                    e_v = g * 16 + iota
                    degv = plsc.load_gather(rows_b0, [e_v, degcol])
                    rinv = 1.0 / jnp.maximum(degv, 1.0)

                    def ocol(k, carry2):
                        fk = jnp.full((16,), k, _i32)
                        av = plsc.load_gather(rows_b0, [e_v, fk])
                        bv = plsc.load_gather(emb_b, [e_v, fk])
                        plsc.store_scatter(rows_b0, [e_v, fk], bv + av * rinv)
                        return carry2

                    lax.fori_loop(0, EMB, ocol, 0)

                pltpu.sync_copy(rows_b0, out_hbm.at[q, pl.ds(rb, 128)])

        plsc.subcore_barrier()


_mp_kernel = pl.kernel(
    _mp_body,
    out_type=jax.ShapeDtypeStruct((NQ, QPAD, CW), _f32),
    mesh=plsc.VectorSubcoreMesh(core_axis_name="c", subcore_axis_name="s"),
    compiler_params=pltpu.CompilerParams(needs_layout_passes=False),
    scratch_types=[
        pltpu.VMEM_SHARED((QPAD, CW), _f32),
        pltpu.VMEM((32,), _i32),
        pltpu.VMEM((16, CW), _f32),
        pltpu.VMEM((EC, CW), _f32),
        pltpu.VMEM((EC, CW), _f32),
        pltpu.VMEM((EC, CW), _f32),
        pltpu.VMEM((EC, CW), _f32),
        pltpu.VMEM((128, CW), _f32),
        pltpu.VMEM((EC,), _i32),
        pltpu.VMEM((EC,), _i32),
        pltpu.VMEM((EC,), _i32),
        pltpu.VMEM((EC,), _i32),
        pltpu.VMEM((EC,), _i32),
        pltpu.VMEM((EC,), _i32),
        pltpu.VMEM((1, EC), _i32),
        pltpu.SemaphoreType.DMA,
        pltpu.SemaphoreType.DMA,
        pltpu.SemaphoreType.DMA,
        pltpu.SemaphoreType.DMA,
    ],
)


def _lstm_loss_kernel(x_ref, h_ref, c_ref, wih_ref, whh_ref, b_ref, cand_ref,
                      out_ref):
    i = pl.program_id(0)
    x = x_ref[...]
    h = h_ref[...]
    gates = (lax.dot_general(x, wih_ref[...], (((1,), (1,)), ((), ())),
                             preferred_element_type=jnp.float32)
             + lax.dot_general(h, whh_ref[...], (((1,), (1,)), ((), ())),
                               preferred_element_type=jnp.float32)
             + b_ref[...])
    i_g = jax.nn.sigmoid(gates[:, 0:EMBP])
    f_g = jax.nn.sigmoid(gates[:, EMBP:2 * EMBP])
    g_g = jnp.tanh(gates[:, 2 * EMBP:3 * EMBP])
    o_g = jax.nn.sigmoid(gates[:, 3 * EMBP:4 * EMBP])
    c_new = f_g * c_ref[...] + i_g * g_g
    h_new = o_g * jnp.tanh(c_new)
    score = jnp.sum(h_new[:, None, :] * cand_ref[...], axis=-1)
    col = lax.broadcasted_iota(jnp.int32, score.shape, 1)
    score = jnp.where(col < N_CAND, score, -1e30)
    m = jnp.max(score, axis=-1, keepdims=True)
    lse = m[:, 0] + jnp.log(jnp.sum(jnp.exp(score - m), axis=-1))
    part = jnp.sum(score[:, 0] - lse).reshape(1, 1)

    @pl.when(i == 0)
    def _():
        out_ref[...] = jnp.zeros((1, 1), jnp.float32)

    out_ref[...] += part


def _lstm_loss(x, h, c, wih, whh, b, cand):
    nblk = B_SEED // BLK
    out = pl.pallas_call(
        _lstm_loss_kernel,
        grid=(nblk,),
        in_specs=[
            pl.BlockSpec((BLK, EMBP), lambda i: (i, 0)),
            pl.BlockSpec((BLK, EMBP), lambda i: (i, 0)),
            pl.BlockSpec((BLK, EMBP), lambda i: (i, 0)),
            pl.BlockSpec((4 * EMBP, EMBP), lambda i: (0, 0)),
            pl.BlockSpec((4 * EMBP, EMBP), lambda i: (0, 0)),
            pl.BlockSpec((1, 4 * EMBP), lambda i: (0, 0)),
            pl.BlockSpec((BLK, N_CANDP, EMBP), lambda i: (i, 0, 0)),
        ],
        out_specs=pl.BlockSpec((1, 1), lambda i: (0, 0)),
        out_shape=jax.ShapeDtypeStruct((1, 1), jnp.float32),
    )(x, h, c, wih, whh, b, cand)
    return out[0, 0] * (-1.0 / B_SEED)


def _pad_gate_w(w):
    w = w.reshape(4, EMB, EMB)
    w = jnp.pad(w, ((0, 0), (0, EMBP - EMB), (0, EMBP - EMB)))
    return w.reshape(4 * EMBP, EMBP)


def kernel(node_emb, cx, cat_table, W_ih, W_hh, b_ih, b_hh,
           edge_src, edge_dst, edge_cat, seed_idx, ns_idx):
    emb_pad = jnp.zeros((NP, CW), _f32)
    emb_pad = emb_pad.at[:N_NODES, :EMB].set(node_emb)
    emb_pad = emb_pad.at[:N_NODES, DEGC].set(1.0)
    src_p = jnp.pad(edge_src.astype(_i32), (0, 256))
    dst_p = jnp.pad(edge_dst.astype(_i32), (0, 256))
    cat_p = jnp.pad(edge_cat.astype(_i32), (0, 256))
    qb = jnp.array(QB, _i32)
    eoffs = jnp.searchsorted(edge_dst, qb).astype(_i32)
    offs = jnp.zeros((32,), _i32).at[0:NQ + 1].set(eoffs).at[16:16 + NQ + 1].set(qb)

    ctab_pad = jnp.zeros((32, CW), _f32)
    ctab_pad = ctab_pad.at[:CAT_NUM, :EMB].set(cat_table.astype(_f32))
    ctab_pad = ctab_pad.at[:CAT_NUM, DEGC].set(1.0)

    node2 = _mp_kernel(emb_pad, src_p, dst_p, cat_p, ctab_pad, offs)
    node_emb2 = jnp.concatenate(
        [node2[i, :QB[i + 1] - QB[i], :EMB] for i in range(NQ)], axis=0)

    x = node_emb2[seed_idx]
    h = node_emb[seed_idx]
    c = cx[seed_idx]
    cand = node_emb2[ns_idx + USER_NUM]

    pad = ((0, 0), (0, EMBP - EMB))
    x = jnp.pad(x, pad)
    h = jnp.pad(h, pad)
    c = jnp.pad(c, pad)
    cand = jnp.pad(cand, ((0, 0), (0, N_CANDP - N_CAND), (0, EMBP - EMB)))
    wih = _pad_gate_w(W_ih)
    whh = _pad_gate_w(W_hh)
    b = (b_ih + b_hh).reshape(4, EMB)
    b = jnp.pad(b, ((0, 0), (0, EMBP - EMB))).reshape(1, 4 * EMBP)
    return _lstm_loss(x, h, c, wih, whh, b, cand)


# speedup vs baseline: 4.8561x; 1.8777x over previous
import functools

import jax
import jax.numpy as jnp
from jax import lax
from jax.experimental import pallas as pl
from jax.experimental.pallas import tpu as pltpu
from jax.experimental.pallas import tpu_sc as plsc

N_NODES = 50000
USER_NUM = 40000
CAT_NUM = 18
EMB = 100
EMBP = 128
B_SEED = 4096
N_CAND = 9
N_CANDP = 16
BLK = 512

NQ = 10
QB = (0, 5008, 10016, 15024, 20032, 25040, 30048, 35056, 40064, 45072, 50000)
QPAD = 5120
DUMP = 5050
CW = 128
DEGC = 100
EC = 128
NP = 50176
N_OUT_CHUNKS = 40

_i32 = jnp.int32
_f32 = jnp.float32


def _mp_body(emb_hbm, src_hbm, dst_hbm, ecat_hbm, ctab_hbm, offs_hbm, out_hbm,
             agg, ctab_sh, offs_v, zero_b, rows_b0, rows_b1, crows_b0,
             crows_b1, emb_b, srci_b0, srci_b1, dst_b0, dst_b1, cat_b0,
             cat_b1, dstloc_b, isem0, isem1, gsem0, gsem1):
    rows_b = (rows_b0, rows_b1)
    crows_b = (crows_b0, crows_b1)
    srci_b = (srci_b0, srci_b1)
    dst_b = (dst_b0, dst_b1)
    cat_b = (cat_b0, cat_b1)
    isem = (isem0, isem1)
    gsem = (gsem0, gsem1)
    c = lax.axis_index("c")
    s = lax.axis_index("s")
    iota = lax.iota(_i32, 16)

    pltpu.sync_copy(offs_hbm, offs_v)

    @pl.when(s == 0)
    def _ctab():
        pltpu.sync_copy(ctab_hbm, ctab_sh)

    def ext(j):
        v = offs_v[pl.ds((j // 16) * 16, 16)]
        return jnp.max(jnp.where(iota == (j % 16), v, 0))

    zvec = jnp.zeros((16,), _f32)

    def zrow(i, carry):
        for cc in range(CW // 16):
            zero_b[i, pl.ds(cc * 16, 16)] = zvec
        return carry

    lax.fori_loop(0, 16, zrow, 0)

    for r in range(NQ // 2):
        q = (NQ // 2) * c + r
        qbase = pl.multiple_of(ext(16 + q), 8)

        zb = s * (QPAD // 16)
        for j in range(QPAD // 16 // 16):
            pltpu.sync_copy(zero_b, agg.at[pl.ds(zb + j * 16, 16)])
        plsc.subcore_barrier()

        lo = ext(q)
        hi = ext(q + 1)
        cnt = hi - lo
        my_lo = lo + (cnt * s) // 16
        my_hi = lo + (cnt * (s + 1)) // 16
        my_lo_al = pl.multiple_of((my_lo // 8) * 8, 8)
        nch = (my_hi - my_lo_al + EC - 1) // EC
        ncl = jnp.maximum(nch, 1)

        def cbase(j):
            return my_lo_al + jnp.minimum(j, ncl - 1) * EC

        def issue_idx(j, b):
            base = pl.multiple_of(cbase(j), 8)
            pltpu.async_copy(src_hbm.at[pl.ds(base, EC)], srci_b[b], isem[b])
            pltpu.async_copy(dst_hbm.at[pl.ds(base, EC)], dst_b[b], isem[b])
            pltpu.async_copy(ecat_hbm.at[pl.ds(base, EC)], cat_b[b], isem[b])

        def wait_idx(b):
            pltpu.make_async_copy(src_hbm.at[pl.ds(0, EC)], srci_b[b],
                                  isem[b]).wait()
            pltpu.make_async_copy(dst_hbm.at[pl.ds(0, EC)], dst_b[b],
                                  isem[b]).wait()
            pltpu.make_async_copy(ecat_hbm.at[pl.ds(0, EC)], cat_b[b],
                                  isem[b]).wait()

        def issue_gather(b):
            pltpu.async_copy(emb_hbm.at[srci_b[b]], rows_b[b], gsem[b])
            pltpu.sync_copy(ctab_sh.at[cat_b[b]], crows_b[b])

        def wait_gather(b):
            pltpu.make_async_copy(emb_hbm.at[srci_b[b]], rows_b[b],
                                  gsem[b]).wait()

        def calc_dstloc(j, b):
            base = my_lo_al + j * EC
            for g in range(EC // 16):
                e_v = g * 16 + iota
                pos = base + e_v
                dstv = dst_b[b][pl.ds(g * 16, 16)]
                valid = (pos >= my_lo) & (pos < my_hi)
                dstloc_b[0, pl.ds(g * 16, 16)] = jnp.where(
                    valid, dstv - qbase, DUMP)

        def mul_scatter(b):
            def mrow(i, carry2):
                for cc in range(CW // 16):
                    sl = pl.ds(cc * 16, 16)
                    rows_b[b][i, sl] = rows_b[b][i, sl] * crows_b[b][i, sl]
                return carry2

            lax.fori_loop(0, EC, mrow, 0)
            pltpu.sync_copy(rows_b[b], agg.at[dstloc_b.at[0]], add=True)

        issue_idx(0, 0)
        issue_idx(1, 1)
        wait_idx(0)
        issue_gather(0)

        def pipebody(g, carry):
            j0 = 2 * g
            j1 = j0 + 1
            wait_idx(1)
            issue_gather(1)
            wait_gather(0)
            calc_dstloc(j0, 0)
            issue_idx(j0 + 2, 0)
            mul_scatter(0)
            wait_idx(0)
            wait_gather(1)
            issue_gather(0)
            calc_dstloc(j1, 1)
            issue_idx(j1 + 2, 1)
            mul_scatter(1)
            return carry

        npair = (ncl + 1) // 2
        lax.fori_loop(0, npair - 1, pipebody, 0)

        t2 = 2 * npair - 2
        wait_idx(1)
        issue_gather(1)
        wait_gather(0)
        calc_dstloc(t2, 0)
        mul_scatter(0)
        wait_gather(1)
        calc_dstloc(t2 + 1, 1)
        mul_scatter(1)
        plsc.subcore_barrier()

        for j in range(3):
            m = s + 16 * j

            @pl.when(m < N_OUT_CHUNKS)
            def _chunk(m=m):
                rb = m * 128
                pltpu.sync_copy(agg.at[pl.ds(rb, 128)], rows_b0)
                pltpu.sync_copy(emb_hbm.at[pl.ds(qbase + rb, 128)], emb_b)
                degcol = jnp.full((16,), DEGC, _i32)
                for g in range(8):
                    e_v = g * 16 + iota
                    degv = plsc.load_gather(rows_b0, [e_v, degcol])
                    rinv = 1.0 / jnp.maximum(degv, 1.0)

                    def ocol(k, carry2):
                        fk = jnp.full((16,), k, _i32)
                        av = plsc.load_gather(rows_b0, [e_v, fk])
                        bv = plsc.load_gather(emb_b, [e_v, fk])
                        plsc.store_scatter(rows_b0, [e_v, fk], bv + av * rinv)
                        return carry2

                    lax.fori_loop(0, EMB, ocol, 0)

                pltpu.sync_copy(rows_b0, out_hbm.at[q, pl.ds(rb, 128)])

        plsc.subcore_barrier()


_mp_kernel = pl.kernel(
    _mp_body,
    out_type=jax.ShapeDtypeStruct((NQ, QPAD, CW), _f32),
    mesh=plsc.VectorSubcoreMesh(core_axis_name="c", subcore_axis_name="s"),
    compiler_params=pltpu.CompilerParams(needs_layout_passes=False),
    scratch_types=[
        pltpu.VMEM_SHARED((QPAD, CW), _f32),
        pltpu.VMEM_SHARED((32, CW), _f32),
        pltpu.VMEM((32,), _i32),
        pltpu.VMEM((16, CW), _f32),
        pltpu.VMEM((EC, CW), _f32),
        pltpu.VMEM((EC, CW), _f32),
        pltpu.VMEM((EC, CW), _f32),
        pltpu.VMEM((EC, CW), _f32),
        pltpu.VMEM((128, CW), _f32),
        pltpu.VMEM((EC,), _i32),
        pltpu.VMEM((EC,), _i32),
        pltpu.VMEM((EC,), _i32),
        pltpu.VMEM((EC,), _i32),
        pltpu.VMEM((EC,), _i32),
        pltpu.VMEM((EC,), _i32),
        pltpu.VMEM((1, EC), _i32),
        pltpu.SemaphoreType.DMA,
        pltpu.SemaphoreType.DMA,
        pltpu.SemaphoreType.DMA,
        pltpu.SemaphoreType.DMA,
    ],
)


def _lstm_loss_kernel(x_ref, h_ref, c_ref, wih_ref, whh_ref, b_ref, cand_ref,
                      out_ref):
    i = pl.program_id(0)
    x = x_ref[...]
    h = h_ref[...]
    gates = (lax.dot_general(x, wih_ref[...], (((1,), (1,)), ((), ())),
                             preferred_element_type=jnp.float32)
             + lax.dot_general(h, whh_ref[...], (((1,), (1,)), ((), ())),
                               preferred_element_type=jnp.float32)
             + b_ref[...])
    i_g = jax.nn.sigmoid(gates[:, 0:EMBP])
    f_g = jax.nn.sigmoid(gates[:, EMBP:2 * EMBP])
    g_g = jnp.tanh(gates[:, 2 * EMBP:3 * EMBP])
    o_g = jax.nn.sigmoid(gates[:, 3 * EMBP:4 * EMBP])
    c_new = f_g * c_ref[...] + i_g * g_g
    h_new = o_g * jnp.tanh(c_new)
    score = jnp.sum(h_new[:, None, :] * cand_ref[...], axis=-1)
    col = lax.broadcasted_iota(jnp.int32, score.shape, 1)
    score = jnp.where(col < N_CAND, score, -1e30)
    m = jnp.max(score, axis=-1, keepdims=True)
    lse = m[:, 0] + jnp.log(jnp.sum(jnp.exp(score - m), axis=-1))
    part = jnp.sum(score[:, 0] - lse).reshape(1, 1)

    @pl.when(i == 0)
    def _():
        out_ref[...] = jnp.zeros((1, 1), jnp.float32)

    out_ref[...] += part


def _lstm_loss(x, h, c, wih, whh, b, cand):
    nblk = B_SEED // BLK
    out = pl.pallas_call(
        _lstm_loss_kernel,
        grid=(nblk,),
        in_specs=[
            pl.BlockSpec((BLK, EMBP), lambda i: (i, 0)),
            pl.BlockSpec((BLK, EMBP), lambda i: (i, 0)),
            pl.BlockSpec((BLK, EMBP), lambda i: (i, 0)),
            pl.BlockSpec((4 * EMBP, EMBP), lambda i: (0, 0)),
            pl.BlockSpec((4 * EMBP, EMBP), lambda i: (0, 0)),
            pl.BlockSpec((1, 4 * EMBP), lambda i: (0, 0)),
            pl.BlockSpec((BLK, N_CANDP, EMBP), lambda i: (i, 0, 0)),
        ],
        out_specs=pl.BlockSpec((1, 1), lambda i: (0, 0)),
        out_shape=jax.ShapeDtypeStruct((1, 1), jnp.float32),
    )(x, h, c, wih, whh, b, cand)
    return out[0, 0] * (-1.0 / B_SEED)


def _pad_gate_w(w):
    w = w.reshape(4, EMB, EMB)
    w = jnp.pad(w, ((0, 0), (0, EMBP - EMB), (0, EMBP - EMB)))
    return w.reshape(4 * EMBP, EMBP)


def kernel(node_emb, cx, cat_table, W_ih, W_hh, b_ih, b_hh,
           edge_src, edge_dst, edge_cat, seed_idx, ns_idx):
    emb_pad = jnp.zeros((NP, CW), _f32)
    emb_pad = emb_pad.at[:N_NODES, :EMB].set(node_emb)
    emb_pad = emb_pad.at[:N_NODES, DEGC].set(1.0)
    src_p = jnp.pad(edge_src.astype(_i32), (0, 256))
    dst_p = jnp.pad(edge_dst.astype(_i32), (0, 256))
    cat_p = jnp.pad(edge_cat.astype(_i32), (0, 256))
    qb = jnp.array(QB, _i32)
    eoffs = jnp.searchsorted(edge_dst, qb).astype(_i32)
    offs = jnp.zeros((32,), _i32).at[0:NQ + 1].set(eoffs).at[16:16 + NQ + 1].set(qb)

    ctab_pad = jnp.zeros((32, CW), _f32)
    ctab_pad = ctab_pad.at[:CAT_NUM, :EMB].set(cat_table.astype(_f32))
    ctab_pad = ctab_pad.at[:CAT_NUM, DEGC].set(1.0)

    node2 = _mp_kernel(emb_pad, src_p, dst_p, cat_p, ctab_pad, offs)
    node_emb2 = jnp.concatenate(
        [node2[i, :QB[i + 1] - QB[i], :EMB] for i in range(NQ)], axis=0)

    x = node_emb2[seed_idx]
    h = node_emb[seed_idx]
    c = cx[seed_idx]
    cand = node_emb2[ns_idx + USER_NUM]

    pad = ((0, 0), (0, EMBP - EMB))
    x = jnp.pad(x, pad)
    h = jnp.pad(h, pad)
    c = jnp.pad(c, pad)
    cand = jnp.pad(cand, ((0, 0), (0, N_CANDP - N_CAND), (0, EMBP - EMB)))
    wih = _pad_gate_w(W_ih)
    whh = _pad_gate_w(W_hh)
    b = (b_ih + b_hh).reshape(4, EMB)
    b = jnp.pad(b, ((0, 0), (0, EMBP - EMB))).reshape(1, 4 * EMBP)
    return _lstm_loss(x, h, c, wih, whh, b, cand)
